# baseline (speedup 1.0000x reference)
import functools

import jax
import jax.numpy as jnp
from jax import lax
from jax.experimental import pallas as pl
from jax.experimental.pallas import tpu as pltpu
from jax.experimental.pallas import tpu_sc as plsc


def _sc_info():
    try:
        info = plsc.get_sparse_core_info()
        return info.num_cores, info.num_subcores
    except Exception:
        return 2, 16


def _sc_gather(Eh, rvh, Wu, bs, bo, u_idx, r_idx, v_idx):
    B = u_idx.shape[0]
    D = Eh.shape[1]
    NC, NS = _sc_info()
    NW = NC * NS
    bpw = B // NW
    f32 = jnp.float32
    mesh = plsc.VectorSubcoreMesh(core_axis_name="c", subcore_axis_name="s")

    @functools.partial(
        pl.kernel,
        mesh=mesh,
        compiler_params=pltpu.CompilerParams(use_tc_tiling_on_sc=False),
        out_type=(
            jax.ShapeDtypeStruct((B, D), f32),
            jax.ShapeDtypeStruct((B, D), f32),
            jax.ShapeDtypeStruct((B, D), f32),
            jax.ShapeDtypeStruct((B, D), f32),
            jax.ShapeDtypeStruct((B,), f32),
            jax.ShapeDtypeStruct((B,), f32),
        ),
        scratch_types=[
            pltpu.VMEM((bpw,), jnp.int32),
            pltpu.VMEM((bpw,), jnp.int32),
            pltpu.VMEM((bpw,), jnp.int32),
            pltpu.VMEM((bpw, D), f32),
            pltpu.VMEM((bpw, D), f32),
            pltpu.VMEM((bpw, D), f32),
            pltpu.VMEM((bpw, D), f32),
            pltpu.VMEM((bpw,), f32),
            pltpu.VMEM((bpw,), f32),
            pltpu.SemaphoreType.DMA,
        ],
    )
    def gather_k(Eh_h, rvh_h, Wu_h, bs_h, bo_h, ui_h, ri_h, vi_h,
                 u_o, v_o, Ru_o, rv_o, bsu_o, bov_o,
                 ui_v, ri_v, vi_v, u_v, v_v, Ru_v, rv_v, bsu_v, bov_v, sem):
        wid = lax.axis_index("s") * NC + lax.axis_index("c")
        base = wid * bpw
        pltpu.sync_copy(ui_h.at[pl.ds(base, bpw)], ui_v)
        pltpu.sync_copy(ri_h.at[pl.ds(base, bpw)], ri_v)
        pltpu.sync_copy(vi_h.at[pl.ds(base, bpw)], vi_v)
        copies = [
            pltpu.async_copy(Eh_h.at[ui_v], u_v, sem),
            pltpu.async_copy(Eh_h.at[vi_v], v_v, sem),
            pltpu.async_copy(Wu_h.at[ri_v], Ru_v, sem),
            pltpu.async_copy(rvh_h.at[ri_v], rv_v, sem),
            pltpu.async_copy(bs_h.at[ui_v], bsu_v, sem),
            pltpu.async_copy(bo_h.at[vi_v], bov_v, sem),
        ]
        for c in copies:
            c.wait()
        pltpu.sync_copy(u_v, u_o.at[pl.ds(base, bpw)])
        pltpu.sync_copy(v_v, v_o.at[pl.ds(base, bpw)])
        pltpu.sync_copy(Ru_v, Ru_o.at[pl.ds(base, bpw)])
        pltpu.sync_copy(rv_v, rv_o.at[pl.ds(base, bpw)])
        pltpu.sync_copy(bsu_v, bsu_o.at[pl.ds(base, bpw)])
        pltpu.sync_copy(bov_v, bov_o.at[pl.ds(base, bpw)])

    return gather_k(Eh, rvh, Wu, bs, bo, u_idx, r_idx, v_idx)


def _math_body(u_ref, v_ref, Ru_ref, rv_ref, bsu_ref, bov_ref, o_ref):
    u = u_ref[...]
    v = v_ref[...]
    Ru = Ru_ref[...]
    rvh_e = rv_ref[...]

    def _norm(x):
        return jnp.sqrt(jnp.sum(x * x, axis=-1, keepdims=True))

    def _proj(x):
        n = _norm(x)
        return jnp.where(n >= 1.0, x / (n - 1e-5), x)

    def _artanh(x):
        return 0.5 * jnp.log((1.0 + x) / (1.0 - x))

    def _p_sum(x, y):
        sqx = jnp.clip(jnp.sum(x * x, axis=-1, keepdims=True), 0.0, 1.0 - 1e-5)
        sqy = jnp.clip(jnp.sum(y * y, axis=-1, keepdims=True), 0.0, 1.0 - 1e-5)
        dxy = jnp.sum(x * y, axis=-1, keepdims=True)
        num = (1.0 + 2.0 * dxy + sqy) * x + (1.0 - sqx) * y
        den = 1.0 + 2.0 * dxy + sqx * sqy
        return num / den

    u = _proj(u)
    v = _proj(v)
    rvh_e = _proj(rvh_e)
    nu = jnp.clip(_norm(u), 1e-10, 1.0 - 1e-5)
    u_e = _artanh(nu) * u / nu
    u_W = u_e * Ru
    nw = jnp.clip(_norm(u_W), 1e-10, None)
    u_m = jnp.tanh(nw) * u_W / nw
    v_m = _p_sum(v, rvh_e)
    u_m = _proj(u_m)
    v_m = _proj(v_m)
    d = _p_sum(-u_m, v_m)
    nd = jnp.clip(_norm(d), 1e-10, 1.0 - 1e-5)
    sq = (2.0 * _artanh(nd)) ** 2
    o_ref[...] = -sq + bsu_ref[...] + bov_ref[...]


def _tc_math(u, v, Ru, rvh_e, bsu, bov, interpret=False):
    B, D = u.shape
    BLK = 2048
    spec_d = pl.BlockSpec((BLK, D), lambda i: (i, 0))
    spec_1 = pl.BlockSpec((BLK, 1), lambda i: (i, 0))
    out = pl.pallas_call(
        _math_body,
        grid=(B // BLK,),
        in_specs=[spec_d, spec_d, spec_d, spec_d, spec_1, spec_1],
        out_specs=spec_1,
        out_shape=jax.ShapeDtypeStruct((B, 1), jnp.float32),
        interpret=interpret,
    )(u, v, Ru, rvh_e, bsu.reshape(B, 1), bov.reshape(B, 1))
    return out.reshape(B)


def kernel(Eh, rvh, Wu, bs, bo, u_idx, r_idx, v_idx):
    u_idx = u_idx.astype(jnp.int32)
    r_idx = r_idx.astype(jnp.int32)
    v_idx = v_idx.astype(jnp.int32)
    u, v, Ru, rvh_e, bsu, bov = _sc_gather(Eh, rvh, Wu, bs, bo, u_idx, r_idx, v_idx)
    return _tc_math(u, v, Ru, rvh_e, bsu, bov)


# speedup vs baseline: 1.5393x; 1.5393x over previous
import functools

import jax
import jax.numpy as jnp
from jax import lax
from jax.experimental import pallas as pl
from jax.experimental.pallas import tpu as pltpu
from jax.experimental.pallas import tpu_sc as plsc

f32 = jnp.float32
i32 = jnp.int32


def _sc_info():
    try:
        info = plsc.get_sparse_core_info()
        return info.num_cores, info.num_subcores
    except Exception:
        return 2, 16


def _sc_gather_small(rvh, Wu, bs, bo, r_idx, u_idx, v_idx):
    B = r_idx.shape[0]
    D = rvh.shape[1]
    NC, NS = _sc_info()
    NW = NC * NS
    bpw = B // NW
    mesh = plsc.VectorSubcoreMesh(core_axis_name="c", subcore_axis_name="s")

    @functools.partial(
        pl.kernel,
        mesh=mesh,
        compiler_params=pltpu.CompilerParams(use_tc_tiling_on_sc=False),
        out_type=(
            jax.ShapeDtypeStruct((B, D), f32),
            jax.ShapeDtypeStruct((B, D), f32),
            jax.ShapeDtypeStruct((B,), f32),
            jax.ShapeDtypeStruct((B,), f32),
        ),
        scratch_types=[
            pltpu.VMEM((bpw,), i32),
            pltpu.VMEM((bpw,), i32),
            pltpu.VMEM((bpw,), i32),
            pltpu.VMEM((bpw, D), f32),
            pltpu.VMEM((bpw, D), f32),
            pltpu.VMEM((bpw,), f32),
            pltpu.VMEM((bpw,), f32),
            pltpu.SemaphoreType.DMA,
        ],
    )
    def ka(rvh_h, Wu_h, bs_h, bo_h, ri_h, ui_h, vi_h,
           Ru_o, rv_o, bsu_o, bov_o,
           ri_v, ui_v, vi_v, Ru_v, rv_v, bsu_v, bov_v, sem):
        wid = lax.axis_index("s") * NC + lax.axis_index("c")
        base = wid * bpw
        pltpu.sync_copy(ri_h.at[pl.ds(base, bpw)], ri_v)
        pltpu.sync_copy(ui_h.at[pl.ds(base, bpw)], ui_v)
        pltpu.sync_copy(vi_h.at[pl.ds(base, bpw)], vi_v)
        copies = [
            pltpu.async_copy(Wu_h.at[ri_v], Ru_v, sem),
            pltpu.async_copy(rvh_h.at[ri_v], rv_v, sem),
            pltpu.async_copy(bs_h.at[ui_v], bsu_v, sem),
            pltpu.async_copy(bo_h.at[vi_v], bov_v, sem),
        ]
        for c in copies:
            c.wait()
        pltpu.sync_copy(Ru_v, Ru_o.at[pl.ds(base, bpw)])
        pltpu.sync_copy(rv_v, rv_o.at[pl.ds(base, bpw)])
        pltpu.sync_copy(bsu_v, bsu_o.at[pl.ds(base, bpw)])
        pltpu.sync_copy(bov_v, bov_o.at[pl.ds(base, bpw)])

    return ka(rvh, Wu, bs, bo, r_idx, u_idx, v_idx)


def _sc_gather_entity(Eh, u_idx, v_idx):
    B = u_idx.shape[0]
    D = Eh.shape[1]
    NC, NS = _sc_info()
    NW = NC * NS
    bpw = B // NW
    mesh = plsc.VectorSubcoreMesh(core_axis_name="c", subcore_axis_name="s")

    CH = bpw // 2

    @functools.partial(
        pl.kernel,
        mesh=mesh,
        compiler_params=pltpu.CompilerParams(use_tc_tiling_on_sc=True),
        out_type=(
            jax.ShapeDtypeStruct((B, D), f32),
            jax.ShapeDtypeStruct((B, D), f32),
        ),
        scratch_types=[
            pltpu.VMEM((bpw,), i32),
            pltpu.VMEM((bpw,), i32),
            pltpu.VMEM((CH, D), f32),
            pltpu.VMEM((CH, D), f32),
            pltpu.SemaphoreType.DMA,
            pltpu.SemaphoreType.DMA,
        ],
    )
    def kb(Eh_h, ui_h, vi_h, u_o, v_o, ui_v, vi_v, u_rows, v_rows, sem, sem2):
        wid = lax.axis_index("s") * NC + lax.axis_index("c")
        base = wid * bpw
        pltpu.sync_copy(ui_h.at[pl.ds(base, bpw)], ui_v)
        pltpu.sync_copy(vi_h.at[pl.ds(base, bpw)], vi_v)

        for h in range(2):
            h0 = h * CH

            def body(g, carry):
                g0 = g * 16
                idx16u = ui_v[pl.ds(h0 + g0, 16)]
                idx16v = vi_v[pl.ds(h0 + g0, 16)]
                for k in range(16):
                    su = idx16u[k]
                    sv = idx16v[k]
                    pltpu.make_async_copy(
                        Eh_h.at[pl.ds(su, 1)],
                        u_rows.at[pl.ds(g0 + k, 1)], sem,
                    ).start()
                    pltpu.make_async_copy(
                        Eh_h.at[pl.ds(sv, 1)],
                        v_rows.at[pl.ds(g0 + k, 1)], sem2,
                    ).start()
                return carry

            lax.fori_loop(0, CH // 16, body, 0)
            pltpu.make_async_copy(Eh_h.at[pl.ds(0, CH)], u_rows, sem).wait()
            pltpu.make_async_copy(Eh_h.at[pl.ds(0, CH)], v_rows, sem2).wait()
            pltpu.sync_copy(u_rows, u_o.at[pl.ds(base + h0, CH)])
            pltpu.sync_copy(v_rows, v_o.at[pl.ds(base + h0, CH)])

    return kb(Eh, u_idx, v_idx)


def _math_body(u_ref, v_ref, Ru_ref, rv_ref, bsu_ref, bov_ref, o_ref):
    u = u_ref[...]
    v = v_ref[...]
    Ru = Ru_ref[...]
    rvh_e = rv_ref[...]

    def _norm(x):
        return jnp.sqrt(jnp.sum(x * x, axis=-1, keepdims=True))

    def _proj(x):
        n = _norm(x)
        return jnp.where(n >= 1.0, x / (n - 1e-5), x)

    def _artanh(x):
        return 0.5 * jnp.log((1.0 + x) / (1.0 - x))

    def _p_sum(x, y):
        sqx = jnp.clip(jnp.sum(x * x, axis=-1, keepdims=True), 0.0, 1.0 - 1e-5)
        sqy = jnp.clip(jnp.sum(y * y, axis=-1, keepdims=True), 0.0, 1.0 - 1e-5)
        dxy = jnp.sum(x * y, axis=-1, keepdims=True)
        num = (1.0 + 2.0 * dxy + sqy) * x + (1.0 - sqx) * y
        den = 1.0 + 2.0 * dxy + sqx * sqy
        return num / den

    u = _proj(u)
    v = _proj(v)
    rvh_e = _proj(rvh_e)
    nu = jnp.clip(_norm(u), 1e-10, 1.0 - 1e-5)
    u_e = _artanh(nu) * u / nu
    u_W = u_e * Ru
    nw = jnp.clip(_norm(u_W), 1e-10, None)
    u_m = jnp.tanh(nw) * u_W / nw
    v_m = _p_sum(v, rvh_e)
    u_m = _proj(u_m)
    v_m = _proj(v_m)
    d = _p_sum(-u_m, v_m)
    nd = jnp.clip(_norm(d), 1e-10, 1.0 - 1e-5)
    sq = (2.0 * _artanh(nd)) ** 2
    o_ref[...] = -sq + bsu_ref[...] + bov_ref[...]


def _tc_math(u, v, Ru, rvh_e, bsu, bov, interpret=False):
    B, D = u.shape
    BLK = 2048
    spec_d = pl.BlockSpec((BLK, D), lambda i: (i, 0))
    spec_1 = pl.BlockSpec((BLK, 1), lambda i: (i, 0))
    out = pl.pallas_call(
        _math_body,
        grid=(B // BLK,),
        in_specs=[spec_d, spec_d, spec_d, spec_d, spec_1, spec_1],
        out_specs=spec_1,
        out_shape=jax.ShapeDtypeStruct((B, 1), jnp.float32),
        interpret=interpret,
    )(u, v, Ru, rvh_e, bsu.reshape(B, 1), bov.reshape(B, 1))
    return out.reshape(B)


def kernel(Eh, rvh, Wu, bs, bo, u_idx, r_idx, v_idx):
    u_idx = u_idx.astype(i32)
    r_idx = r_idx.astype(i32)
    v_idx = v_idx.astype(i32)
    u, v = _sc_gather_entity(Eh, u_idx, v_idx)
    Ru, rvh_e, bsu, bov = _sc_gather_small(rvh, Wu, bs, bo, r_idx, u_idx, v_idx)
    return _tc_math(u, v, Ru, rvh_e, bsu, bov)


# speedup vs baseline: 5.6843x; 3.6926x over previous
import functools

import jax
import jax.numpy as jnp
from jax import lax
from jax.experimental import pallas as pl
from jax.experimental.pallas import tpu as pltpu
from jax.experimental.pallas import tpu_sc as plsc

f32 = jnp.float32
i32 = jnp.int32


def _sc_info():
    try:
        info = plsc.get_sparse_core_info()
        return info.num_cores, info.num_subcores
    except Exception:
        return 2, 16


def _sc_gather_small(rvh, Wu, bs, bo, r_idx, u_idx, v_idx):
    B = r_idx.shape[0]
    D = rvh.shape[1]
    NC, NS = _sc_info()
    NW = NC * NS
    bpw = B // NW
    mesh = plsc.VectorSubcoreMesh(core_axis_name="c", subcore_axis_name="s")

    @functools.partial(
        pl.kernel,
        mesh=mesh,
        compiler_params=pltpu.CompilerParams(use_tc_tiling_on_sc=False),
        out_type=(
            jax.ShapeDtypeStruct((B, D), f32),
            jax.ShapeDtypeStruct((B, D), f32),
            jax.ShapeDtypeStruct((B,), f32),
            jax.ShapeDtypeStruct((B,), f32),
        ),
        scratch_types=[
            pltpu.VMEM((bpw,), i32),
            pltpu.VMEM((bpw,), i32),
            pltpu.VMEM((bpw,), i32),
            pltpu.VMEM((bpw, D), f32),
            pltpu.VMEM((bpw, D), f32),
            pltpu.VMEM((bpw,), f32),
            pltpu.VMEM((bpw,), f32),
            pltpu.SemaphoreType.DMA,
        ],
    )
    def ka(rvh_h, Wu_h, bs_h, bo_h, ri_h, ui_h, vi_h,
           Ru_o, rv_o, bsu_o, bov_o,
           ri_v, ui_v, vi_v, Ru_v, rv_v, bsu_v, bov_v, sem):
        wid = lax.axis_index("s") * NC + lax.axis_index("c")
        base = wid * bpw
        pltpu.sync_copy(ri_h.at[pl.ds(base, bpw)], ri_v)
        pltpu.sync_copy(ui_h.at[pl.ds(base, bpw)], ui_v)
        pltpu.sync_copy(vi_h.at[pl.ds(base, bpw)], vi_v)
        copies = [
            pltpu.async_copy(Wu_h.at[ri_v], Ru_v, sem),
            pltpu.async_copy(rvh_h.at[ri_v], rv_v, sem),
            pltpu.async_copy(bs_h.at[ui_v], bsu_v, sem),
            pltpu.async_copy(bo_h.at[vi_v], bov_v, sem),
        ]
        for c in copies:
            c.wait()
        pltpu.sync_copy(Ru_v, Ru_o.at[pl.ds(base, bpw)])
        pltpu.sync_copy(rv_v, rv_o.at[pl.ds(base, bpw)])
        pltpu.sync_copy(bsu_v, bsu_o.at[pl.ds(base, bpw)])
        pltpu.sync_copy(bov_v, bov_o.at[pl.ds(base, bpw)])

    return ka(rvh, Wu, bs, bo, r_idx, u_idx, v_idx)


def _sc_gather_entity(EhT, u_idx, v_idx):
    D, N = EhT.shape
    B = u_idx.shape[0]
    NC, NS = _sc_info()
    NW = NC * NS
    bpw = B // NW
    mesh = plsc.VectorSubcoreMesh(core_axis_name="c", subcore_axis_name="s")

    @functools.partial(
        pl.kernel,
        mesh=mesh,
        compiler_params=pltpu.CompilerParams(use_tc_tiling_on_sc=True),
        out_type=(
            jax.ShapeDtypeStruct((D, B), f32),
            jax.ShapeDtypeStruct((D, B), f32),
        ),
        scratch_types=[
            pltpu.VMEM((bpw,), i32),
            pltpu.VMEM((bpw,), i32),
            pltpu.VMEM((D, bpw), f32),
            pltpu.VMEM((D, bpw), f32),
            pltpu.SemaphoreType.DMA,
            pltpu.SemaphoreType.DMA,
        ],
    )
    def kb(EhT_h, ui_h, vi_h, u_o, v_o, ui_v, vi_v, u_cols, v_cols, sem, sem2):
        wid = lax.axis_index("s") * NC + lax.axis_index("c")
        base = wid * bpw
        pltpu.sync_copy(ui_h.at[pl.ds(base, bpw)], ui_v)
        pltpu.sync_copy(vi_h.at[pl.ds(base, bpw)], vi_v)

        def body(g, carry):
            g0 = g * 16
            idx16u = ui_v[pl.ds(g0, 16)]
            idx16v = vi_v[pl.ds(g0, 16)]
            for k in range(16):
                su = idx16u[k]
                sv = idx16v[k]
                pltpu.make_async_copy(
                    EhT_h.at[:, pl.ds(su, 1)],
                    u_cols.at[:, pl.ds(g0 + k, 1)], sem,
                ).start()
                pltpu.make_async_copy(
                    EhT_h.at[:, pl.ds(sv, 1)],
                    v_cols.at[:, pl.ds(g0 + k, 1)], sem2,
                ).start()
            return carry

        lax.fori_loop(0, bpw // 16, body, 0)
        pltpu.make_async_copy(EhT_h.at[:, pl.ds(0, bpw)], u_cols, sem).wait()
        pltpu.make_async_copy(EhT_h.at[:, pl.ds(0, bpw)], v_cols, sem2).wait()
        pltpu.sync_copy(u_cols, u_o.at[:, pl.ds(base, bpw)])
        pltpu.sync_copy(v_cols, v_o.at[:, pl.ds(base, bpw)])

    return kb(EhT, u_idx, v_idx)


def _math_body(uT_ref, vT_ref, Ru_ref, rv_ref, bsu_ref, bov_ref, o_ref):
    uT = uT_ref[...]
    vT = vT_ref[...]
    RuT = Ru_ref[...].T
    rvT = rv_ref[...].T

    def _norm(x):
        return jnp.sqrt(jnp.sum(x * x, axis=0, keepdims=True))

    def _proj(x):
        n = _norm(x)
        return jnp.where(n >= 1.0, x / (n - 1e-5), x)

    def _artanh(x):
        return 0.5 * jnp.log((1.0 + x) / (1.0 - x))

    def _p_sum(x, y):
        sqx = jnp.clip(jnp.sum(x * x, axis=0, keepdims=True), 0.0, 1.0 - 1e-5)
        sqy = jnp.clip(jnp.sum(y * y, axis=0, keepdims=True), 0.0, 1.0 - 1e-5)
        dxy = jnp.sum(x * y, axis=0, keepdims=True)
        num = (1.0 + 2.0 * dxy + sqy) * x + (1.0 - sqx) * y
        den = 1.0 + 2.0 * dxy + sqx * sqy
        return num / den

    u = _proj(uT)
    v = _proj(vT)
    rvh_e = _proj(rvT)
    nu = jnp.clip(_norm(u), 1e-10, 1.0 - 1e-5)
    u_e = _artanh(nu) * u / nu
    u_W = u_e * RuT
    nw = jnp.clip(_norm(u_W), 1e-10, None)
    u_m = jnp.tanh(nw) * u_W / nw
    v_m = _p_sum(v, rvh_e)
    u_m = _proj(u_m)
    v_m = _proj(v_m)
    d = _p_sum(-u_m, v_m)
    nd = jnp.clip(_norm(d), 1e-10, 1.0 - 1e-5)
    sq = (2.0 * _artanh(nd)) ** 2
    res = -lax.squeeze(sq, (0,)) + bsu_ref[...] + bov_ref[...]
    o_ref[...] = res


def _tc_math(uT, vT, Ru, rvh_e, bsu, bov, interpret=False):
    D, B = uT.shape
    BLK = 2048
    spec_t = pl.BlockSpec((D, BLK), lambda i: (0, i))
    spec_d = pl.BlockSpec((BLK, D), lambda i: (i, 0))
    spec_1 = pl.BlockSpec((BLK,), lambda i: (i,))
    out = pl.pallas_call(
        _math_body,
        grid=(B // BLK,),
        in_specs=[spec_t, spec_t, spec_d, spec_d, spec_1, spec_1],
        out_specs=spec_1,
        out_shape=jax.ShapeDtypeStruct((B,), jnp.float32),
        interpret=interpret,
    )(uT, vT, Ru, rvh_e, bsu, bov)
    return out


def kernel(Eh, rvh, Wu, bs, bo, u_idx, r_idx, v_idx):
    u_idx = u_idx.astype(i32)
    r_idx = r_idx.astype(i32)
    v_idx = v_idx.astype(i32)
    uT = jnp.take(Eh, u_idx, axis=0).T
    vT = jnp.take(Eh, v_idx, axis=0).T
    Ru, rvh_e, bsu, bov = _sc_gather_small(rvh, Wu, bs, bo, r_idx, u_idx, v_idx)
    return _tc_math(uT, vT, Ru, rvh_e, bsu, bov)
